# baseline (speedup 1.0000x reference)
import jax
import jax.numpy as jnp
from jax import lax
from jax.experimental import pallas as pl
from jax.experimental.pallas import tpu as pltpu
from jax.experimental.pallas import tpu_sc as plsc

N_NODES, D_FEAT = 100000, 128
N0, N1, B = 30976, 2816, 256
INTERNAL, NUM_CLASSES = 128, 64

NC, NS = 2, 16
NW = NC * NS
CHUNK = 128
N0_CHUNKS = N0 // CHUNK
N1_CHUNKS = N1 // CHUNK
B_CHUNKS = B // CHUNK

K_BLK = 1408
PIECES = (11, 66, 165)


def _sc_gather_l1_piece(features, src_nodes, d2s, d2d, start, n_chunks,
                        with_dst):
    rounds = -(-n_chunks // NW)
    n_tasks = rounds + (1 if with_dst else 0)

    def body(features_, src_nodes_, d2s_, d2d_, *rest):
        if with_dst:
            src_out, dst_out = rest[0], rest[1]
            scratch = rest[2:]
        else:
            src_out, dst_out = rest[0], None
            scratch = rest[1:]
        j_all, idx_all, rows2, sem_j, sem_i, g0, g1, s0, s1 = scratch
        gsem = (g0, g1)
        ssem = (s0, s1)
        wid = lax.axis_index("s") * NC + lax.axis_index("c")

        tasks = []
        for t in range(rounds):
            c = wid + t * NW
            cid = jnp.where(c < n_chunks, c, c - n_chunks)
            base = pl.multiple_of(cid * CHUNK, CHUNK)
            tasks.append((d2s_, start * CHUNK + base, src_out, base))
        if with_dst:
            dc = jnp.where(wid < N1_CHUNKS, wid, wid - N1_CHUNKS)
            dbase = pl.multiple_of(dc * CHUNK, CHUNK)
            tasks.append((d2d_, dbase, dst_out, dbase))

        jcopies = [
            pltpu.make_async_copy(ih.at[pl.ds(bi, CHUNK)], j_all.at[t], sem_j)
            for t, (ih, bi, _, _) in enumerate(tasks)
        ]
        for cp in jcopies:
            cp.start()
        for cp in jcopies:
            cp.wait()

        icopies = [
            pltpu.make_async_copy(src_nodes_.at[j_all.at[t]], idx_all.at[t],
                                  sem_i)
            for t in range(n_tasks)
        ]
        for cp in icopies:
            cp.start()
        for cp in icopies:
            cp.wait()

        gets = [
            pltpu.make_async_copy(features_.at[idx_all.at[t]],
                                  rows2.at[t % 2], gsem[t % 2])
            for t in range(n_tasks)
        ]
        puts = [
            pltpu.make_async_copy(rows2.at[t % 2],
                                  oh.at[pl.ds(bo, CHUNK)], ssem[t % 2])
            for t, (_, _, oh, bo) in enumerate(tasks)
        ]
        gets[0].start()
        for t in range(1, n_tasks):
            if t >= 2:
                puts[t - 2].wait()
            gets[t].start()
            gets[t - 1].wait()
            puts[t - 1].start()
        gets[n_tasks - 1].wait()
        puts[n_tasks - 1].start()
        if n_tasks >= 2:
            puts[n_tasks - 2].wait()
        puts[n_tasks - 1].wait()

    mesh = plsc.VectorSubcoreMesh(core_axis_name="c", subcore_axis_name="s")
    out_type = [jax.ShapeDtypeStruct((n_chunks * CHUNK, D_FEAT), jnp.float32)]
    if with_dst:
        out_type.append(jax.ShapeDtypeStruct((N1, D_FEAT), jnp.float32))

    return pl.kernel(
        body,
        out_type=out_type,
        mesh=mesh,
        scratch_types=[
            pltpu.VMEM((n_tasks, CHUNK), jnp.int32),
            pltpu.VMEM((n_tasks, CHUNK), jnp.int32),
            pltpu.VMEM((2, CHUNK, D_FEAT), jnp.float32),
            pltpu.SemaphoreType.DMA,
            pltpu.SemaphoreType.DMA,
            pltpu.SemaphoreType.DMA,
            pltpu.SemaphoreType.DMA,
            pltpu.SemaphoreType.DMA,
            pltpu.SemaphoreType.DMA,
        ],
    )(features, src_nodes, d2s, d2d)


def _sc_gather_l2_body(h1, d2s, d2d, src_out, dst_out, j_v, rows_v, sem):
    wid = lax.axis_index("s") * NC + lax.axis_index("c")

    def do_chunk(cid, idx_hbm, out_hbm):
        base = pl.multiple_of(cid * CHUNK, CHUNK)
        pltpu.sync_copy(idx_hbm.at[pl.ds(base, CHUNK)], j_v)
        pltpu.async_copy(h1.at[j_v], rows_v, sem).wait()
        pltpu.sync_copy(rows_v, out_hbm.at[pl.ds(base, CHUNK)])

    @pl.when(wid < N1_CHUNKS)
    def _():
        do_chunk(wid, d2s, src_out)

    @pl.when((wid >= N1_CHUNKS) & (wid < N1_CHUNKS + B_CHUNKS))
    def _():
        do_chunk(wid - N1_CHUNKS, d2d, dst_out)


def _sc_gather_l2(h1, d2s, d2d):
    mesh = plsc.VectorSubcoreMesh(core_axis_name="c", subcore_axis_name="s")
    return pl.kernel(
        _sc_gather_l2_body,
        out_type=[
            jax.ShapeDtypeStruct((N1, INTERNAL), jnp.float32),
            jax.ShapeDtypeStruct((B, INTERNAL), jnp.float32),
        ],
        mesh=mesh,
        scratch_types=[
            pltpu.VMEM((CHUNK,), jnp.int32),
            pltpu.VMEM((CHUNK, INTERNAL), jnp.float32),
            pltpu.SemaphoreType.DMA,
        ],
    )(h1, d2s, d2d)


def _tc_piece(dm1, src_piece, acc_in, step_off, n_steps, tail):
    def body(*refs):
        if tail:
            dm_ref, sf_ref, acc_in_ref, df_ref, w1_ref, out_ref, acc_ref = refs
        elif acc_in is not None:
            dm_ref, sf_ref, acc_in_ref, out_ref, acc_ref = refs
        else:
            dm_ref, sf_ref, out_ref, acc_ref = refs
            acc_in_ref = None
        k = pl.program_id(0)

        @pl.when(k == 0)
        def _():
            if acc_in_ref is None:
                acc_ref[...] = jnp.zeros_like(acc_ref)
            else:
                acc_ref[...] = acc_in_ref[...]

        acc_ref[...] += jnp.dot(dm_ref[...], sf_ref[...],
                                preferred_element_type=jnp.float32)

        @pl.when(k == n_steps - 1)
        def _():
            if tail:
                w1 = w1_ref[...]
                h = (jnp.dot(acc_ref[...], w1[:D_FEAT, :],
                             preferred_element_type=jnp.float32)
                     + jnp.dot(df_ref[...], w1[D_FEAT:, :],
                               preferred_element_type=jnp.float32))
                out_ref[...] = jnp.maximum(h, 0.0)
            else:
                out_ref[...] = acc_ref[...]

    in_specs = [
        pl.BlockSpec((N1, K_BLK), lambda k: (0, k + step_off)),
        pl.BlockSpec((K_BLK, D_FEAT), lambda k: (k, 0)),
    ]
    args = [dm1, src_piece]
    if acc_in is not None:
        in_specs.append(pl.BlockSpec((N1, D_FEAT), lambda k: (0, 0)))
        args.append(acc_in)
    if tail:
        dst_feat, W1 = tail
        in_specs.append(pl.BlockSpec((N1, D_FEAT), lambda k: (0, 0)))
        in_specs.append(pl.BlockSpec((2 * D_FEAT, INTERNAL), lambda k: (0, 0)))
        args.extend([dst_feat, W1])

    return pl.pallas_call(
        body,
        grid=(n_steps,),
        in_specs=in_specs,
        out_specs=pl.BlockSpec((N1, D_FEAT), lambda k: (0, 0)),
        out_shape=jax.ShapeDtypeStruct((N1, D_FEAT), jnp.float32),
        scratch_shapes=[pltpu.VMEM((N1, D_FEAT), jnp.float32)],
        compiler_params=pltpu.CompilerParams(
            dimension_semantics=("arbitrary",),
        ),
    )(*args)


def _tc_layer2_body(dm2_ref, sf2_ref, df2_ref, w2_ref, wc_ref, out_ref):
    agg = jnp.dot(dm2_ref[...], sf2_ref[...],
                  preferred_element_type=jnp.float32)
    w2 = w2_ref[...]
    h = jnp.maximum(
        jnp.dot(agg, w2[:INTERNAL, :], preferred_element_type=jnp.float32)
        + jnp.dot(df2_ref[...], w2[INTERNAL:, :],
                  preferred_element_type=jnp.float32),
        0.0)
    logits = jnp.dot(h, wc_ref[...], preferred_element_type=jnp.float32)
    m = jnp.max(logits, axis=-1, keepdims=True)
    e = jnp.exp(logits - m)
    out_ref[...] = e / jnp.sum(e, axis=-1, keepdims=True)


def _tc_layer2(dm2, src_feat2, dst_feat2, W2, Wc):
    return pl.pallas_call(
        _tc_layer2_body,
        out_shape=jax.ShapeDtypeStruct((B, NUM_CLASSES), jnp.float32),
    )(dm2, src_feat2, dst_feat2, W2, Wc)


def kernel(features, src_nodes, dstsrc2src_l1, dstsrc2dst_l1, dif_mat_l1,
           dstsrc2src_l2, dstsrc2dst_l2, dif_mat_l2, W1, W2, Wc):
    n_pieces = len(PIECES)
    starts = [sum(PIECES[:i]) for i in range(n_pieces)]

    src_pieces = []
    dst_feat1 = None
    for i, (start, n_chunks) in enumerate(zip(starts, PIECES)):
        last = i == n_pieces - 1
        res = _sc_gather_l1_piece(features, src_nodes, dstsrc2src_l1,
                                  dstsrc2dst_l1, start, n_chunks,
                                  with_dst=last)
        src_pieces.append(res[0])
        if last:
            dst_feat1 = res[1]

    acc = None
    for i, (start, n_chunks) in enumerate(zip(starts, PIECES)):
        last = i == n_pieces - 1
        tail = (dst_feat1, W1) if last else None
        acc = _tc_piece(dm1=dif_mat_l1, src_piece=src_pieces[i], acc_in=acc,
                        step_off=start * CHUNK // K_BLK,
                        n_steps=n_chunks * CHUNK // K_BLK, tail=tail)
    h1 = acc

    src_feat2, dst_feat2 = _sc_gather_l2(h1, dstsrc2src_l2, dstsrc2dst_l2)
    return _tc_layer2(dif_mat_l2, src_feat2, dst_feat2, W2, Wc)


# speedup vs baseline: 1.0199x; 1.0199x over previous
import jax
import jax.numpy as jnp
from jax import lax
from jax.experimental import pallas as pl
from jax.experimental.pallas import tpu as pltpu
from jax.experimental.pallas import tpu_sc as plsc

N_NODES, D_FEAT = 100000, 128
N0, N1, B = 30976, 2816, 256
INTERNAL, NUM_CLASSES = 128, 64

NC, NS = 2, 16
NW = NC * NS
CHUNK = 128
N0_CHUNKS = N0 // CHUNK
N1_CHUNKS = N1 // CHUNK
B_CHUNKS = B // CHUNK

K_BLK = 1408
PIECES = (22, 66, 154)


def _sc_gather_l1_piece(features, src_nodes, d2s, d2d, start, n_chunks,
                        with_dst):
    rounds = -(-n_chunks // NW)
    n_tasks = rounds + (1 if with_dst else 0)

    def body(features_, src_nodes_, d2s_, d2d_, *rest):
        if with_dst:
            src_out, dst_out = rest[0], rest[1]
            scratch = rest[2:]
        else:
            src_out, dst_out = rest[0], None
            scratch = rest[1:]
        j_all, idx_all, rows2, sem_j, sem_i, g0, g1, s0, s1 = scratch
        gsem = (g0, g1)
        ssem = (s0, s1)
        wid = lax.axis_index("s") * NC + lax.axis_index("c")

        tasks = []
        for t in range(rounds):
            c = wid + t * NW
            cid = jnp.where(c < n_chunks, c, c - n_chunks)
            base = pl.multiple_of(cid * CHUNK, CHUNK)
            tasks.append((d2s_, start * CHUNK + base, src_out, base))
        if with_dst:
            dc = jnp.where(wid < N1_CHUNKS, wid, wid - N1_CHUNKS)
            dbase = pl.multiple_of(dc * CHUNK, CHUNK)
            tasks.append((d2d_, dbase, dst_out, dbase))

        jcopies = [
            pltpu.make_async_copy(ih.at[pl.ds(bi, CHUNK)], j_all.at[t], sem_j)
            for t, (ih, bi, _, _) in enumerate(tasks)
        ]
        for cp in jcopies:
            cp.start()
        for cp in jcopies:
            cp.wait()

        icopies = [
            pltpu.make_async_copy(src_nodes_.at[j_all.at[t]], idx_all.at[t],
                                  sem_i)
            for t in range(n_tasks)
        ]
        for cp in icopies:
            cp.start()
        for cp in icopies:
            cp.wait()

        gets = [
            pltpu.make_async_copy(features_.at[idx_all.at[t]],
                                  rows2.at[t % 2], gsem[t % 2])
            for t in range(n_tasks)
        ]
        puts = [
            pltpu.make_async_copy(rows2.at[t % 2],
                                  oh.at[pl.ds(bo, CHUNK)], ssem[t % 2])
            for t, (_, _, oh, bo) in enumerate(tasks)
        ]
        gets[0].start()
        for t in range(1, n_tasks):
            if t >= 2:
                puts[t - 2].wait()
            gets[t].start()
            gets[t - 1].wait()
            puts[t - 1].start()
        gets[n_tasks - 1].wait()
        puts[n_tasks - 1].start()
        if n_tasks >= 2:
            puts[n_tasks - 2].wait()
        puts[n_tasks - 1].wait()

    mesh = plsc.VectorSubcoreMesh(core_axis_name="c", subcore_axis_name="s")
    out_type = [jax.ShapeDtypeStruct((n_chunks * CHUNK, D_FEAT), jnp.float32)]
    if with_dst:
        out_type.append(jax.ShapeDtypeStruct((N1, D_FEAT), jnp.float32))

    return pl.kernel(
        body,
        out_type=out_type,
        mesh=mesh,
        scratch_types=[
            pltpu.VMEM((n_tasks, CHUNK), jnp.int32),
            pltpu.VMEM((n_tasks, CHUNK), jnp.int32),
            pltpu.VMEM((2, CHUNK, D_FEAT), jnp.float32),
            pltpu.SemaphoreType.DMA,
            pltpu.SemaphoreType.DMA,
            pltpu.SemaphoreType.DMA,
            pltpu.SemaphoreType.DMA,
            pltpu.SemaphoreType.DMA,
            pltpu.SemaphoreType.DMA,
        ],
    )(features, src_nodes, d2s, d2d)


def _sc_gather_l2_body(h1, d2s, d2d, src_out, dst_out, j_v, rows_v, sem):
    wid = lax.axis_index("s") * NC + lax.axis_index("c")

    def do_chunk(cid, idx_hbm, out_hbm):
        base = pl.multiple_of(cid * CHUNK, CHUNK)
        pltpu.sync_copy(idx_hbm.at[pl.ds(base, CHUNK)], j_v)
        pltpu.async_copy(h1.at[j_v], rows_v, sem).wait()
        pltpu.sync_copy(rows_v, out_hbm.at[pl.ds(base, CHUNK)])

    @pl.when(wid < N1_CHUNKS)
    def _():
        do_chunk(wid, d2s, src_out)

    @pl.when((wid >= N1_CHUNKS) & (wid < N1_CHUNKS + B_CHUNKS))
    def _():
        do_chunk(wid - N1_CHUNKS, d2d, dst_out)


def _sc_gather_l2(h1, d2s, d2d):
    mesh = plsc.VectorSubcoreMesh(core_axis_name="c", subcore_axis_name="s")
    return pl.kernel(
        _sc_gather_l2_body,
        out_type=[
            jax.ShapeDtypeStruct((N1, INTERNAL), jnp.float32),
            jax.ShapeDtypeStruct((B, INTERNAL), jnp.float32),
        ],
        mesh=mesh,
        scratch_types=[
            pltpu.VMEM((CHUNK,), jnp.int32),
            pltpu.VMEM((CHUNK, INTERNAL), jnp.float32),
            pltpu.SemaphoreType.DMA,
        ],
    )(h1, d2s, d2d)


def _tc_piece(dm1, src_piece, acc_in, step_off, n_steps, tail):
    def body(*refs):
        if tail:
            dm_ref, sf_ref, acc_in_ref, df_ref, w1_ref, out_ref, acc_ref = refs
        elif acc_in is not None:
            dm_ref, sf_ref, acc_in_ref, out_ref, acc_ref = refs
        else:
            dm_ref, sf_ref, out_ref, acc_ref = refs
            acc_in_ref = None
        k = pl.program_id(0)

        @pl.when(k == 0)
        def _():
            if acc_in_ref is None:
                acc_ref[...] = jnp.zeros_like(acc_ref)
            else:
                acc_ref[...] = acc_in_ref[...]

        acc_ref[...] += jnp.dot(dm_ref[...], sf_ref[...],
                                preferred_element_type=jnp.float32)

        @pl.when(k == n_steps - 1)
        def _():
            if tail:
                w1 = w1_ref[...]
                h = (jnp.dot(acc_ref[...], w1[:D_FEAT, :],
                             preferred_element_type=jnp.float32)
                     + jnp.dot(df_ref[...], w1[D_FEAT:, :],
                               preferred_element_type=jnp.float32))
                out_ref[...] = jnp.maximum(h, 0.0)
            else:
                out_ref[...] = acc_ref[...]

    in_specs = [
        pl.BlockSpec((N1, K_BLK), lambda k: (0, k + step_off)),
        pl.BlockSpec((K_BLK, D_FEAT), lambda k: (k, 0)),
    ]
    args = [dm1, src_piece]
    if acc_in is not None:
        in_specs.append(pl.BlockSpec((N1, D_FEAT), lambda k: (0, 0)))
        args.append(acc_in)
    if tail:
        dst_feat, W1 = tail
        in_specs.append(pl.BlockSpec((N1, D_FEAT), lambda k: (0, 0)))
        in_specs.append(pl.BlockSpec((2 * D_FEAT, INTERNAL), lambda k: (0, 0)))
        args.extend([dst_feat, W1])

    return pl.pallas_call(
        body,
        grid=(n_steps,),
        in_specs=in_specs,
        out_specs=pl.BlockSpec((N1, D_FEAT), lambda k: (0, 0)),
        out_shape=jax.ShapeDtypeStruct((N1, D_FEAT), jnp.float32),
        scratch_shapes=[pltpu.VMEM((N1, D_FEAT), jnp.float32)],
        compiler_params=pltpu.CompilerParams(
            dimension_semantics=("arbitrary",),
        ),
    )(*args)


def _tc_layer2_body(dm2_ref, sf2_ref, df2_ref, w2_ref, wc_ref, out_ref):
    agg = jnp.dot(dm2_ref[...], sf2_ref[...],
                  preferred_element_type=jnp.float32)
    w2 = w2_ref[...]
    h = jnp.maximum(
        jnp.dot(agg, w2[:INTERNAL, :], preferred_element_type=jnp.float32)
        + jnp.dot(df2_ref[...], w2[INTERNAL:, :],
                  preferred_element_type=jnp.float32),
        0.0)
    logits = jnp.dot(h, wc_ref[...], preferred_element_type=jnp.float32)
    m = jnp.max(logits, axis=-1, keepdims=True)
    e = jnp.exp(logits - m)
    out_ref[...] = e / jnp.sum(e, axis=-1, keepdims=True)


def _tc_layer2(dm2, src_feat2, dst_feat2, W2, Wc):
    return pl.pallas_call(
        _tc_layer2_body,
        out_shape=jax.ShapeDtypeStruct((B, NUM_CLASSES), jnp.float32),
    )(dm2, src_feat2, dst_feat2, W2, Wc)


def kernel(features, src_nodes, dstsrc2src_l1, dstsrc2dst_l1, dif_mat_l1,
           dstsrc2src_l2, dstsrc2dst_l2, dif_mat_l2, W1, W2, Wc):
    n_pieces = len(PIECES)
    starts = [sum(PIECES[:i]) for i in range(n_pieces)]

    src_pieces = []
    dst_feat1 = None
    for i, (start, n_chunks) in enumerate(zip(starts, PIECES)):
        last = i == n_pieces - 1
        res = _sc_gather_l1_piece(features, src_nodes, dstsrc2src_l1,
                                  dstsrc2dst_l1, start, n_chunks,
                                  with_dst=last)
        src_pieces.append(res[0])
        if last:
            dst_feat1 = res[1]

    acc = None
    for i, (start, n_chunks) in enumerate(zip(starts, PIECES)):
        last = i == n_pieces - 1
        tail = (dst_feat1, W1) if last else None
        acc = _tc_piece(dm1=dif_mat_l1, src_piece=src_pieces[i], acc_in=acc,
                        step_off=start * CHUNK // K_BLK,
                        n_steps=n_chunks * CHUNK // K_BLK, tail=tail)
    h1 = acc

    src_feat2, dst_feat2 = _sc_gather_l2(h1, dstsrc2src_l2, dstsrc2dst_l2)
    return _tc_layer2(dif_mat_l2, src_feat2, dst_feat2, W2, Wc)


# speedup vs baseline: 1.0306x; 1.0105x over previous
import jax
import jax.numpy as jnp
from jax import lax
from jax.experimental import pallas as pl
from jax.experimental.pallas import tpu as pltpu
from jax.experimental.pallas import tpu_sc as plsc

N_NODES, D_FEAT = 100000, 128
N0, N1, B = 30976, 2816, 256
INTERNAL, NUM_CLASSES = 128, 64

NC, NS = 2, 16
NW = NC * NS
CHUNK = 128
N0_CHUNKS = N0 // CHUNK
N1_CHUNKS = N1 // CHUNK
B_CHUNKS = B // CHUNK
TS = 88

K_BLK = 1408
PIECES = (22, 66, 154)


def _sc_gather_l1_piece(features, src_nodes, d2s, d2d, start, n_chunks,
                        with_dst):
    piece_rows = n_chunks * CHUNK
    assert piece_rows % (NW * TS) == 0
    rounds = piece_rows // (NW * TS)
    n_tasks = rounds + (1 if with_dst else 0)

    def body(features_, src_nodes_, d2s_, d2d_, *rest):
        if with_dst:
            src_out, dst_out = rest[0], rest[1]
            scratch = rest[2:]
        else:
            src_out, dst_out = rest[0], None
            scratch = rest[1:]
        j_all, idx_all, rows2, sem_j, sem_i, g0, g1, s0, s1 = scratch
        gsem = (g0, g1)
        ssem = (s0, s1)
        wid = lax.axis_index("s") * NC + lax.axis_index("c")

        tasks = []
        for t in range(rounds):
            base = pl.multiple_of((wid * rounds + t) * TS, TS)
            tasks.append((d2s_, start * CHUNK + base, src_out, base))
        if with_dst:
            dbase = pl.multiple_of(wid * TS, TS)
            tasks.append((d2d_, dbase, dst_out, dbase))

        jcopies = [
            pltpu.make_async_copy(ih.at[pl.ds(bi, TS)], j_all.at[t], sem_j)
            for t, (ih, bi, _, _) in enumerate(tasks)
        ]
        for cp in jcopies:
            cp.start()
        for cp in jcopies:
            cp.wait()

        icopies = [
            pltpu.make_async_copy(src_nodes_.at[j_all.at[t]], idx_all.at[t],
                                  sem_i)
            for t in range(n_tasks)
        ]
        for cp in icopies:
            cp.start()
        for cp in icopies:
            cp.wait()

        gets = [
            pltpu.make_async_copy(features_.at[idx_all.at[t]],
                                  rows2.at[t % 2], gsem[t % 2])
            for t in range(n_tasks)
        ]
        puts = [
            pltpu.make_async_copy(rows2.at[t % 2],
                                  oh.at[pl.ds(bo, TS)], ssem[t % 2])
            for t, (_, _, oh, bo) in enumerate(tasks)
        ]
        gets[0].start()
        for t in range(1, n_tasks):
            if t >= 2:
                puts[t - 2].wait()
            gets[t].start()
            gets[t - 1].wait()
            puts[t - 1].start()
        gets[n_tasks - 1].wait()
        puts[n_tasks - 1].start()
        if n_tasks >= 2:
            puts[n_tasks - 2].wait()
        puts[n_tasks - 1].wait()

    mesh = plsc.VectorSubcoreMesh(core_axis_name="c", subcore_axis_name="s")
    out_type = [jax.ShapeDtypeStruct((n_chunks * CHUNK, D_FEAT), jnp.float32)]
    if with_dst:
        out_type.append(jax.ShapeDtypeStruct((N1, D_FEAT), jnp.float32))

    return pl.kernel(
        body,
        out_type=out_type,
        mesh=mesh,
        scratch_types=[
            pltpu.VMEM((n_tasks, TS), jnp.int32),
            pltpu.VMEM((n_tasks, TS), jnp.int32),
            pltpu.VMEM((2, TS, D_FEAT), jnp.float32),
            pltpu.SemaphoreType.DMA,
            pltpu.SemaphoreType.DMA,
            pltpu.SemaphoreType.DMA,
            pltpu.SemaphoreType.DMA,
            pltpu.SemaphoreType.DMA,
            pltpu.SemaphoreType.DMA,
        ],
    )(features, src_nodes, d2s, d2d)


DTS = B // NW


def _sc_gather_l2_body(h1, d2s, d2d, src_out, dst_out, j_v, jd_v, rows_v,
                       rowsd_v, sem, sem2):
    wid = lax.axis_index("s") * NC + lax.axis_index("c")

    base = pl.multiple_of(wid * TS, TS)
    pltpu.sync_copy(d2s.at[pl.ds(base, TS)], j_v)
    dbase = pl.multiple_of(wid * DTS, DTS)
    pltpu.sync_copy(d2d.at[pl.ds(dbase, DTS)], jd_v)
    g1 = pltpu.make_async_copy(h1.at[j_v], rows_v, sem)
    g2 = pltpu.make_async_copy(h1.at[jd_v], rowsd_v, sem2)
    g1.start()
    g2.start()
    g1.wait()
    pltpu.sync_copy(rows_v, src_out.at[pl.ds(base, TS)])
    g2.wait()
    pltpu.sync_copy(rowsd_v, dst_out.at[pl.ds(dbase, DTS)])


def _sc_gather_l2(h1, d2s, d2d):
    mesh = plsc.VectorSubcoreMesh(core_axis_name="c", subcore_axis_name="s")
    return pl.kernel(
        _sc_gather_l2_body,
        out_type=[
            jax.ShapeDtypeStruct((N1, INTERNAL), jnp.float32),
            jax.ShapeDtypeStruct((B, INTERNAL), jnp.float32),
        ],
        mesh=mesh,
        scratch_types=[
            pltpu.VMEM((TS,), jnp.int32),
            pltpu.VMEM((DTS,), jnp.int32),
            pltpu.VMEM((TS, INTERNAL), jnp.float32),
            pltpu.VMEM((DTS, INTERNAL), jnp.float32),
            pltpu.SemaphoreType.DMA,
            pltpu.SemaphoreType.DMA,
        ],
    )(h1, d2s, d2d)


def _tc_piece(dm1, src_piece, acc_in, step_off, n_steps, tail):
    def body(*refs):
        if tail:
            dm_ref, sf_ref, acc_in_ref, df_ref, w1_ref, out_ref, acc_ref = refs
        elif acc_in is not None:
            dm_ref, sf_ref, acc_in_ref, out_ref, acc_ref = refs
        else:
            dm_ref, sf_ref, out_ref, acc_ref = refs
            acc_in_ref = None
        k = pl.program_id(0)

        @pl.when(k == 0)
        def _():
            if acc_in_ref is None:
                acc_ref[...] = jnp.zeros_like(acc_ref)
            else:
                acc_ref[...] = acc_in_ref[...]

        acc_ref[...] += jnp.dot(dm_ref[...], sf_ref[...],
                                preferred_element_type=jnp.float32)

        @pl.when(k == n_steps - 1)
        def _():
            if tail:
                w1 = w1_ref[...]
                h = (jnp.dot(acc_ref[...], w1[:D_FEAT, :],
                             preferred_element_type=jnp.float32)
                     + jnp.dot(df_ref[...], w1[D_FEAT:, :],
                               preferred_element_type=jnp.float32))
                out_ref[...] = jnp.maximum(h, 0.0)
            else:
                out_ref[...] = acc_ref[...]

    in_specs = [
        pl.BlockSpec((N1, K_BLK), lambda k: (0, k + step_off)),
        pl.BlockSpec((K_BLK, D_FEAT), lambda k: (k, 0)),
    ]
    args = [dm1, src_piece]
    if acc_in is not None:
        in_specs.append(pl.BlockSpec((N1, D_FEAT), lambda k: (0, 0)))
        args.append(acc_in)
    if tail:
        dst_feat, W1 = tail
        in_specs.append(pl.BlockSpec((N1, D_FEAT), lambda k: (0, 0)))
        in_specs.append(pl.BlockSpec((2 * D_FEAT, INTERNAL), lambda k: (0, 0)))
        args.extend([dst_feat, W1])

    return pl.pallas_call(
        body,
        grid=(n_steps,),
        in_specs=in_specs,
        out_specs=pl.BlockSpec((N1, D_FEAT), lambda k: (0, 0)),
        out_shape=jax.ShapeDtypeStruct((N1, D_FEAT), jnp.float32),
        scratch_shapes=[pltpu.VMEM((N1, D_FEAT), jnp.float32)],
        compiler_params=pltpu.CompilerParams(
            dimension_semantics=("arbitrary",),
        ),
    )(*args)


def _tc_layer2_body(dm2_ref, sf2_ref, df2_ref, w2_ref, wc_ref, out_ref):
    agg = jnp.dot(dm2_ref[...], sf2_ref[...],
                  preferred_element_type=jnp.float32)
    w2 = w2_ref[...]
    h = jnp.maximum(
        jnp.dot(agg, w2[:INTERNAL, :], preferred_element_type=jnp.float32)
        + jnp.dot(df2_ref[...], w2[INTERNAL:, :],
                  preferred_element_type=jnp.float32),
        0.0)
    logits = jnp.dot(h, wc_ref[...], preferred_element_type=jnp.float32)
    m = jnp.max(logits, axis=-1, keepdims=True)
    e = jnp.exp(logits - m)
    out_ref[...] = e / jnp.sum(e, axis=-1, keepdims=True)


def _tc_layer2(dm2, src_feat2, dst_feat2, W2, Wc):
    return pl.pallas_call(
        _tc_layer2_body,
        out_shape=jax.ShapeDtypeStruct((B, NUM_CLASSES), jnp.float32),
    )(dm2, src_feat2, dst_feat2, W2, Wc)


def kernel(features, src_nodes, dstsrc2src_l1, dstsrc2dst_l1, dif_mat_l1,
           dstsrc2src_l2, dstsrc2dst_l2, dif_mat_l2, W1, W2, Wc):
    n_pieces = len(PIECES)
    starts = [sum(PIECES[:i]) for i in range(n_pieces)]

    src_pieces = []
    dst_feat1 = None
    for i, (start, n_chunks) in enumerate(zip(starts, PIECES)):
        last = i == n_pieces - 1
        res = _sc_gather_l1_piece(features, src_nodes, dstsrc2src_l1,
                                  dstsrc2dst_l1, start, n_chunks,
                                  with_dst=last)
        src_pieces.append(res[0])
        if last:
            dst_feat1 = res[1]

    acc = None
    for i, (start, n_chunks) in enumerate(zip(starts, PIECES)):
        last = i == n_pieces - 1
        tail = (dst_feat1, W1) if last else None
        acc = _tc_piece(dm1=dif_mat_l1, src_piece=src_pieces[i], acc_in=acc,
                        step_off=start * CHUNK // K_BLK,
                        n_steps=n_chunks * CHUNK // K_BLK, tail=tail)
    h1 = acc

    src_feat2, dst_feat2 = _sc_gather_l2(h1, dstsrc2src_l2, dstsrc2dst_l2)
    return _tc_layer2(dif_mat_l2, src_feat2, dst_feat2, W2, Wc)


# speedup vs baseline: 1.0503x; 1.0191x over previous
import jax
import jax.numpy as jnp
from jax import lax
from jax.experimental import pallas as pl
from jax.experimental.pallas import tpu as pltpu
from jax.experimental.pallas import tpu_sc as plsc

N_NODES, D_FEAT = 100000, 128
N0, N1, B = 30976, 2816, 256
INTERNAL, NUM_CLASSES = 128, 64

NC, NS = 2, 16
NW = NC * NS
CHUNK = 128
N0_CHUNKS = N0 // CHUNK
N1_CHUNKS = N1 // CHUNK
B_CHUNKS = B // CHUNK
TS = 88

K_BLK = 1408
PIECES = (44, 198)


def _sc_gather_l1_piece(features, src_nodes, d2s, d2d, start, n_chunks,
                        with_dst):
    piece_rows = n_chunks * CHUNK
    assert piece_rows % (NW * TS) == 0
    rounds = piece_rows // (NW * TS)
    n_tasks = rounds + (1 if with_dst else 0)

    def body(features_, src_nodes_, d2s_, d2d_, *rest):
        if with_dst:
            src_out, dst_out = rest[0], rest[1]
            scratch = rest[2:]
        else:
            src_out, dst_out = rest[0], None
            scratch = rest[1:]
        j_all, idx_all, rows2, sem_j, sem_i, g0, g1, s0, s1 = scratch
        gsem = (g0, g1)
        ssem = (s0, s1)
        wid = lax.axis_index("s") * NC + lax.axis_index("c")

        tasks = []
        for t in range(rounds):
            base = pl.multiple_of((wid * rounds + t) * TS, TS)
            tasks.append((d2s_, start * CHUNK + base, src_out, base))
        if with_dst:
            dbase = pl.multiple_of(wid * TS, TS)
            tasks.append((d2d_, dbase, dst_out, dbase))

        jcopies = [
            pltpu.make_async_copy(ih.at[pl.ds(bi, TS)], j_all.at[t], sem_j)
            for t, (ih, bi, _, _) in enumerate(tasks)
        ]
        for cp in jcopies:
            cp.start()
        for cp in jcopies:
            cp.wait()

        icopies = [
            pltpu.make_async_copy(src_nodes_.at[j_all.at[t]], idx_all.at[t],
                                  sem_i)
            for t in range(n_tasks)
        ]
        for cp in icopies:
            cp.start()
        for cp in icopies:
            cp.wait()

        gets = [
            pltpu.make_async_copy(features_.at[idx_all.at[t]],
                                  rows2.at[t % 2], gsem[t % 2])
            for t in range(n_tasks)
        ]
        puts = [
            pltpu.make_async_copy(rows2.at[t % 2],
                                  oh.at[pl.ds(bo, TS)], ssem[t % 2])
            for t, (_, _, oh, bo) in enumerate(tasks)
        ]
        gets[0].start()
        for t in range(1, n_tasks):
            if t >= 2:
                puts[t - 2].wait()
            gets[t].start()
            gets[t - 1].wait()
            puts[t - 1].start()
        gets[n_tasks - 1].wait()
        puts[n_tasks - 1].start()
        if n_tasks >= 2:
            puts[n_tasks - 2].wait()
        puts[n_tasks - 1].wait()

    mesh = plsc.VectorSubcoreMesh(core_axis_name="c", subcore_axis_name="s")
    out_type = [jax.ShapeDtypeStruct((n_chunks * CHUNK, D_FEAT), jnp.float32)]
    if with_dst:
        out_type.append(jax.ShapeDtypeStruct((N1, D_FEAT), jnp.float32))

    return pl.kernel(
        body,
        out_type=out_type,
        mesh=mesh,
        scratch_types=[
            pltpu.VMEM((n_tasks, TS), jnp.int32),
            pltpu.VMEM((n_tasks, TS), jnp.int32),
            pltpu.VMEM((2, TS, D_FEAT), jnp.float32),
            pltpu.SemaphoreType.DMA,
            pltpu.SemaphoreType.DMA,
            pltpu.SemaphoreType.DMA,
            pltpu.SemaphoreType.DMA,
            pltpu.SemaphoreType.DMA,
            pltpu.SemaphoreType.DMA,
        ],
    )(features, src_nodes, d2s, d2d)


DTS = B // NW


def _sc_gather_l2_body(h1, d2s, d2d, src_out, dst_out, j_v, jd_v, rows_v,
                       rowsd_v, sem, sem2):
    wid = lax.axis_index("s") * NC + lax.axis_index("c")

    base = pl.multiple_of(wid * TS, TS)
    pltpu.sync_copy(d2s.at[pl.ds(base, TS)], j_v)
    dbase = pl.multiple_of(wid * DTS, DTS)
    pltpu.sync_copy(d2d.at[pl.ds(dbase, DTS)], jd_v)
    g1 = pltpu.make_async_copy(h1.at[j_v], rows_v, sem)
    g2 = pltpu.make_async_copy(h1.at[jd_v], rowsd_v, sem2)
    g1.start()
    g2.start()
    g1.wait()
    pltpu.sync_copy(rows_v, src_out.at[pl.ds(base, TS)])
    g2.wait()
    pltpu.sync_copy(rowsd_v, dst_out.at[pl.ds(dbase, DTS)])


def _sc_gather_l2(h1, d2s, d2d):
    mesh = plsc.VectorSubcoreMesh(core_axis_name="c", subcore_axis_name="s")
    return pl.kernel(
        _sc_gather_l2_body,
        out_type=[
            jax.ShapeDtypeStruct((N1, INTERNAL), jnp.float32),
            jax.ShapeDtypeStruct((B, INTERNAL), jnp.float32),
        ],
        mesh=mesh,
        scratch_types=[
            pltpu.VMEM((TS,), jnp.int32),
            pltpu.VMEM((DTS,), jnp.int32),
            pltpu.VMEM((TS, INTERNAL), jnp.float32),
            pltpu.VMEM((DTS, INTERNAL), jnp.float32),
            pltpu.SemaphoreType.DMA,
            pltpu.SemaphoreType.DMA,
        ],
    )(h1, d2s, d2d)


def _tc_piece(dm1, src_piece, acc_in, step_off, n_steps, tail):
    def body(*refs):
        if tail:
            dm_ref, sf_ref, acc_in_ref, df_ref, w1_ref, out_ref, acc_ref = refs
        elif acc_in is not None:
            dm_ref, sf_ref, acc_in_ref, out_ref, acc_ref = refs
        else:
            dm_ref, sf_ref, out_ref, acc_ref = refs
            acc_in_ref = None
        k = pl.program_id(0)

        @pl.when(k == 0)
        def _():
            if acc_in_ref is None:
                acc_ref[...] = jnp.zeros_like(acc_ref)
            else:
                acc_ref[...] = acc_in_ref[...]

        acc_ref[...] += jnp.dot(dm_ref[...], sf_ref[...],
                                preferred_element_type=jnp.float32)

        @pl.when(k == n_steps - 1)
        def _():
            if tail:
                w1 = w1_ref[...]
                h = (jnp.dot(acc_ref[...], w1[:D_FEAT, :],
                             preferred_element_type=jnp.float32)
                     + jnp.dot(df_ref[...], w1[D_FEAT:, :],
                               preferred_element_type=jnp.float32))
                out_ref[...] = jnp.maximum(h, 0.0)
            else:
                out_ref[...] = acc_ref[...]

    in_specs = [
        pl.BlockSpec((N1, K_BLK), lambda k: (0, k + step_off)),
        pl.BlockSpec((K_BLK, D_FEAT), lambda k: (k, 0)),
    ]
    args = [dm1, src_piece]
    if acc_in is not None:
        in_specs.append(pl.BlockSpec((N1, D_FEAT), lambda k: (0, 0)))
        args.append(acc_in)
    if tail:
        dst_feat, W1 = tail
        in_specs.append(pl.BlockSpec((N1, D_FEAT), lambda k: (0, 0)))
        in_specs.append(pl.BlockSpec((2 * D_FEAT, INTERNAL), lambda k: (0, 0)))
        args.extend([dst_feat, W1])

    return pl.pallas_call(
        body,
        grid=(n_steps,),
        in_specs=in_specs,
        out_specs=pl.BlockSpec((N1, D_FEAT), lambda k: (0, 0)),
        out_shape=jax.ShapeDtypeStruct((N1, D_FEAT), jnp.float32),
        scratch_shapes=[pltpu.VMEM((N1, D_FEAT), jnp.float32)],
        compiler_params=pltpu.CompilerParams(
            dimension_semantics=("arbitrary",),
        ),
    )(*args)


def _tc_layer2_body(dm2_ref, sf2_ref, df2_ref, w2_ref, wc_ref, out_ref):
    agg = jnp.dot(dm2_ref[...], sf2_ref[...],
                  preferred_element_type=jnp.float32)
    w2 = w2_ref[...]
    h = jnp.maximum(
        jnp.dot(agg, w2[:INTERNAL, :], preferred_element_type=jnp.float32)
        + jnp.dot(df2_ref[...], w2[INTERNAL:, :],
                  preferred_element_type=jnp.float32),
        0.0)
    logits = jnp.dot(h, wc_ref[...], preferred_element_type=jnp.float32)
    m = jnp.max(logits, axis=-1, keepdims=True)
    e = jnp.exp(logits - m)
    out_ref[...] = e / jnp.sum(e, axis=-1, keepdims=True)


def _tc_layer2(dm2, src_feat2, dst_feat2, W2, Wc):
    return pl.pallas_call(
        _tc_layer2_body,
        out_shape=jax.ShapeDtypeStruct((B, NUM_CLASSES), jnp.float32),
    )(dm2, src_feat2, dst_feat2, W2, Wc)


def kernel(features, src_nodes, dstsrc2src_l1, dstsrc2dst_l1, dif_mat_l1,
           dstsrc2src_l2, dstsrc2dst_l2, dif_mat_l2, W1, W2, Wc):
    n_pieces = len(PIECES)
    starts = [sum(PIECES[:i]) for i in range(n_pieces)]

    src_pieces = []
    dst_feat1 = None
    for i, (start, n_chunks) in enumerate(zip(starts, PIECES)):
        last = i == n_pieces - 1
        res = _sc_gather_l1_piece(features, src_nodes, dstsrc2src_l1,
                                  dstsrc2dst_l1, start, n_chunks,
                                  with_dst=last)
        src_pieces.append(res[0])
        if last:
            dst_feat1 = res[1]

    acc = None
    for i, (start, n_chunks) in enumerate(zip(starts, PIECES)):
        last = i == n_pieces - 1
        tail = (dst_feat1, W1) if last else None
        acc = _tc_piece(dm1=dif_mat_l1, src_piece=src_pieces[i], acc_in=acc,
                        step_off=start * CHUNK // K_BLK,
                        n_steps=n_chunks * CHUNK // K_BLK, tail=tail)
    h1 = acc

    src_feat2, dst_feat2 = _sc_gather_l2(h1, dstsrc2src_l2, dstsrc2dst_l2)
    return _tc_layer2(dif_mat_l2, src_feat2, dst_feat2, W2, Wc)


# speedup vs baseline: 1.0595x; 1.0088x over previous
import jax
import jax.numpy as jnp
from jax import lax
from jax.experimental import pallas as pl
from jax.experimental.pallas import tpu as pltpu
from jax.experimental.pallas import tpu_sc as plsc

N_NODES, D_FEAT = 100000, 128
N0, N1, B = 30976, 2816, 256
INTERNAL, NUM_CLASSES = 128, 64

NC, NS = 2, 16
NW = NC * NS
CHUNK = 128
N0_CHUNKS = N0 // CHUNK
N1_CHUNKS = N1 // CHUNK
B_CHUNKS = B // CHUNK
TS = 88

K_BLK = 1408
PIECES = (22, 220)


def _sc_gather_l1_piece(features, src_nodes, d2s, d2d, start, n_chunks,
                        with_dst):
    piece_rows = n_chunks * CHUNK
    assert piece_rows % (NW * TS) == 0
    rounds = piece_rows // (NW * TS)
    n_tasks = rounds + (1 if with_dst else 0)

    def body(features_, src_nodes_, d2s_, d2d_, *rest):
        if with_dst:
            src_out, dst_out = rest[0], rest[1]
            scratch = rest[2:]
        else:
            src_out, dst_out = rest[0], None
            scratch = rest[1:]
        j_all, idx_all, rows2, sem_j, sem_i, g0, g1, s0, s1 = scratch
        gsem = (g0, g1)
        ssem = (s0, s1)
        wid = lax.axis_index("s") * NC + lax.axis_index("c")

        tasks = []
        for t in range(rounds):
            base = pl.multiple_of((wid * rounds + t) * TS, TS)
            tasks.append((d2s_, start * CHUNK + base, src_out, base))
        if with_dst:
            dbase = pl.multiple_of(wid * TS, TS)
            tasks.append((d2d_, dbase, dst_out, dbase))

        jcopies = [
            pltpu.make_async_copy(ih.at[pl.ds(bi, TS)], j_all.at[t], sem_j)
            for t, (ih, bi, _, _) in enumerate(tasks)
        ]
        for cp in jcopies:
            cp.start()
        for cp in jcopies:
            cp.wait()

        icopies = [
            pltpu.make_async_copy(src_nodes_.at[j_all.at[t]], idx_all.at[t],
                                  sem_i)
            for t in range(n_tasks)
        ]
        for cp in icopies:
            cp.start()
        for cp in icopies:
            cp.wait()

        gets = [
            pltpu.make_async_copy(features_.at[idx_all.at[t]],
                                  rows2.at[t % 2], gsem[t % 2])
            for t in range(n_tasks)
        ]
        puts = [
            pltpu.make_async_copy(rows2.at[t % 2],
                                  oh.at[pl.ds(bo, TS)], ssem[t % 2])
            for t, (_, _, oh, bo) in enumerate(tasks)
        ]
        gets[0].start()
        for t in range(1, n_tasks):
            if t >= 2:
                puts[t - 2].wait()
            gets[t].start()
            gets[t - 1].wait()
            puts[t - 1].start()
        gets[n_tasks - 1].wait()
        puts[n_tasks - 1].start()
        if n_tasks >= 2:
            puts[n_tasks - 2].wait()
        puts[n_tasks - 1].wait()

    mesh = plsc.VectorSubcoreMesh(core_axis_name="c", subcore_axis_name="s")
    out_type = [jax.ShapeDtypeStruct((n_chunks * CHUNK, D_FEAT), jnp.float32)]
    if with_dst:
        out_type.append(jax.ShapeDtypeStruct((N1, D_FEAT), jnp.float32))

    return pl.kernel(
        body,
        out_type=out_type,
        mesh=mesh,
        scratch_types=[
            pltpu.VMEM((n_tasks, TS), jnp.int32),
            pltpu.VMEM((n_tasks, TS), jnp.int32),
            pltpu.VMEM((2, TS, D_FEAT), jnp.float32),
            pltpu.SemaphoreType.DMA,
            pltpu.SemaphoreType.DMA,
            pltpu.SemaphoreType.DMA,
            pltpu.SemaphoreType.DMA,
            pltpu.SemaphoreType.DMA,
            pltpu.SemaphoreType.DMA,
        ],
    )(features, src_nodes, d2s, d2d)


DTS = B // NW


def _sc_gather_l2_body(h1, d2s, d2d, src_out, dst_out, j_v, jd_v, rows_v,
                       rowsd_v, sem, sem2):
    wid = lax.axis_index("s") * NC + lax.axis_index("c")

    base = pl.multiple_of(wid * TS, TS)
    pltpu.sync_copy(d2s.at[pl.ds(base, TS)], j_v)
    dbase = pl.multiple_of(wid * DTS, DTS)
    pltpu.sync_copy(d2d.at[pl.ds(dbase, DTS)], jd_v)
    g1 = pltpu.make_async_copy(h1.at[j_v], rows_v, sem)
    g2 = pltpu.make_async_copy(h1.at[jd_v], rowsd_v, sem2)
    g1.start()
    g2.start()
    g1.wait()
    pltpu.sync_copy(rows_v, src_out.at[pl.ds(base, TS)])
    g2.wait()
    pltpu.sync_copy(rowsd_v, dst_out.at[pl.ds(dbase, DTS)])


def _sc_gather_l2(h1, d2s, d2d):
    mesh = plsc.VectorSubcoreMesh(core_axis_name="c", subcore_axis_name="s")
    return pl.kernel(
        _sc_gather_l2_body,
        out_type=[
            jax.ShapeDtypeStruct((N1, INTERNAL), jnp.float32),
            jax.ShapeDtypeStruct((B, INTERNAL), jnp.float32),
        ],
        mesh=mesh,
        scratch_types=[
            pltpu.VMEM((TS,), jnp.int32),
            pltpu.VMEM((DTS,), jnp.int32),
            pltpu.VMEM((TS, INTERNAL), jnp.float32),
            pltpu.VMEM((DTS, INTERNAL), jnp.float32),
            pltpu.SemaphoreType.DMA,
            pltpu.SemaphoreType.DMA,
        ],
    )(h1, d2s, d2d)


def _tc_piece(dm1, src_piece, acc_in, step_off, n_steps, tail):
    def body(*refs):
        if tail:
            dm_ref, sf_ref, acc_in_ref, df_ref, w1_ref, out_ref, acc_ref = refs
        elif acc_in is not None:
            dm_ref, sf_ref, acc_in_ref, out_ref, acc_ref = refs
        else:
            dm_ref, sf_ref, out_ref, acc_ref = refs
            acc_in_ref = None
        k = pl.program_id(0)

        @pl.when(k == 0)
        def _():
            if acc_in_ref is None:
                acc_ref[...] = jnp.zeros_like(acc_ref)
            else:
                acc_ref[...] = acc_in_ref[...]

        acc_ref[...] += jnp.dot(dm_ref[...], sf_ref[...],
                                preferred_element_type=jnp.float32)

        @pl.when(k == n_steps - 1)
        def _():
            if tail:
                w1 = w1_ref[...]
                h = (jnp.dot(acc_ref[...], w1[:D_FEAT, :],
                             preferred_element_type=jnp.float32)
                     + jnp.dot(df_ref[...], w1[D_FEAT:, :],
                               preferred_element_type=jnp.float32))
                out_ref[...] = jnp.maximum(h, 0.0)
            else:
                out_ref[...] = acc_ref[...]

    in_specs = [
        pl.BlockSpec((N1, K_BLK), lambda k: (0, k + step_off)),
        pl.BlockSpec((K_BLK, D_FEAT), lambda k: (k, 0)),
    ]
    args = [dm1, src_piece]
    if acc_in is not None:
        in_specs.append(pl.BlockSpec((N1, D_FEAT), lambda k: (0, 0)))
        args.append(acc_in)
    if tail:
        dst_feat, W1 = tail
        in_specs.append(pl.BlockSpec((N1, D_FEAT), lambda k: (0, 0)))
        in_specs.append(pl.BlockSpec((2 * D_FEAT, INTERNAL), lambda k: (0, 0)))
        args.extend([dst_feat, W1])

    return pl.pallas_call(
        body,
        grid=(n_steps,),
        in_specs=in_specs,
        out_specs=pl.BlockSpec((N1, D_FEAT), lambda k: (0, 0)),
        out_shape=jax.ShapeDtypeStruct((N1, D_FEAT), jnp.float32),
        scratch_shapes=[pltpu.VMEM((N1, D_FEAT), jnp.float32)],
        compiler_params=pltpu.CompilerParams(
            dimension_semantics=("arbitrary",),
        ),
    )(*args)


def _tc_layer2_body(dm2_ref, sf2_ref, df2_ref, w2_ref, wc_ref, out_ref):
    agg = jnp.dot(dm2_ref[...], sf2_ref[...],
                  preferred_element_type=jnp.float32)
    w2 = w2_ref[...]
    h = jnp.maximum(
        jnp.dot(agg, w2[:INTERNAL, :], preferred_element_type=jnp.float32)
        + jnp.dot(df2_ref[...], w2[INTERNAL:, :],
                  preferred_element_type=jnp.float32),
        0.0)
    logits = jnp.dot(h, wc_ref[...], preferred_element_type=jnp.float32)
    m = jnp.max(logits, axis=-1, keepdims=True)
    e = jnp.exp(logits - m)
    out_ref[...] = e / jnp.sum(e, axis=-1, keepdims=True)


def _tc_layer2(dm2, src_feat2, dst_feat2, W2, Wc):
    return pl.pallas_call(
        _tc_layer2_body,
        out_shape=jax.ShapeDtypeStruct((B, NUM_CLASSES), jnp.float32),
    )(dm2, src_feat2, dst_feat2, W2, Wc)


def kernel(features, src_nodes, dstsrc2src_l1, dstsrc2dst_l1, dif_mat_l1,
           dstsrc2src_l2, dstsrc2dst_l2, dif_mat_l2, W1, W2, Wc):
    n_pieces = len(PIECES)
    starts = [sum(PIECES[:i]) for i in range(n_pieces)]

    src_pieces = []
    dst_feat1 = None
    for i, (start, n_chunks) in enumerate(zip(starts, PIECES)):
        last = i == n_pieces - 1
        res = _sc_gather_l1_piece(features, src_nodes, dstsrc2src_l1,
                                  dstsrc2dst_l1, start, n_chunks,
                                  with_dst=last)
        src_pieces.append(res[0])
        if last:
            dst_feat1 = res[1]

    acc = None
    for i, (start, n_chunks) in enumerate(zip(starts, PIECES)):
        last = i == n_pieces - 1
        tail = (dst_feat1, W1) if last else None
        acc = _tc_piece(dm1=dif_mat_l1, src_piece=src_pieces[i], acc_in=acc,
                        step_off=start * CHUNK // K_BLK,
                        n_steps=n_chunks * CHUNK // K_BLK, tail=tail)
    h1 = acc

    src_feat2, dst_feat2 = _sc_gather_l2(h1, dstsrc2src_l2, dstsrc2dst_l2)
    return _tc_layer2(dif_mat_l2, src_feat2, dst_feat2, W2, Wc)
